# baseline (speedup 1.0000x reference)
import functools

import jax
import jax.numpy as jnp
from jax.experimental import pallas as pl

_TOP_K = 2
_N_EXPERTS = 8
_BLK = 1024


def _gate_body(x_ref, w_ref, idx_ref, tw_ref):
    x = x_ref[...]
    w = w_ref[...]
    logits = jax.lax.dot_general(
        x, w, (((1,), (1,)), ((), ())), preferred_element_type=jnp.float32
    )
    lmax = jnp.max(logits, axis=-1, keepdims=True)
    unnorm = jnp.exp(logits - lmax)
    scores = unnorm / jnp.sum(unnorm, axis=-1, keepdims=True)

    iota = jax.lax.broadcasted_iota(jnp.int32, scores.shape, 1)
    m1 = jnp.max(scores, axis=-1, keepdims=True)
    i1 = jnp.min(
        jnp.where(scores == m1, iota, _N_EXPERTS), axis=-1, keepdims=True
    )
    masked = jnp.where(iota == i1, -jnp.inf, scores)
    m2 = jnp.max(masked, axis=-1, keepdims=True)
    i2 = jnp.min(
        jnp.where(masked == m2, iota, _N_EXPERTS), axis=-1, keepdims=True
    )
    idx_ref[...] = jnp.concatenate([i1, i2], axis=-1)
    tw_ref[...] = jnp.concatenate([m1, m2], axis=-1)


@jax.jit
def kernel(hidden_states, weight):
    bsz, seq_len, h = hidden_states.shape
    n = bsz * seq_len
    x = hidden_states.reshape(n, h)
    grid = (n // _BLK,)
    idx, tw = pl.pallas_call(
        _gate_body,
        grid=grid,
        in_specs=[
            pl.BlockSpec((_BLK, h), lambda i: (i, 0)),
            pl.BlockSpec((_N_EXPERTS, h), lambda i: (0, 0)),
        ],
        out_specs=[
            pl.BlockSpec((_BLK, _TOP_K), lambda i: (i, 0)),
            pl.BlockSpec((_BLK, _TOP_K), lambda i: (i, 0)),
        ],
        out_shape=[
            jax.ShapeDtypeStruct((n, _TOP_K), jnp.int32),
            jax.ShapeDtypeStruct((n, _TOP_K), jnp.float32),
        ],
    )(x, weight)
    return idx, tw


# speedup vs baseline: 2.5264x; 2.5264x over previous
import jax
import jax.numpy as jnp
from jax.experimental import pallas as pl

_TOP_K = 2
_N_EXPERTS = 8
_BLK = 2048


def _gate_body(x_ref, w_ref, idx_ref, tw_ref):
    x = x_ref[...]
    w = w_ref[...]
    logits = jax.lax.dot_general(
        w, x, (((1,), (1,)), ((), ())), preferred_element_type=jnp.float32
    )

    iota = jax.lax.broadcasted_iota(jnp.int32, logits.shape, 0)
    l1 = jnp.max(logits, axis=0, keepdims=True)
    i1 = jnp.min(jnp.where(logits == l1, iota, _N_EXPERTS), axis=0, keepdims=True)
    masked = jnp.where(iota == i1, -jnp.inf, logits)
    l2 = jnp.max(masked, axis=0, keepdims=True)
    i2 = jnp.min(jnp.where(masked == l2, iota, _N_EXPERTS), axis=0, keepdims=True)

    unnorm = jnp.exp(logits - l1)
    denom = jnp.sum(unnorm, axis=0, keepdims=True)
    w1 = jnp.float32(1.0) / denom
    w2 = jnp.exp(l2 - l1) / denom

    idx_ref[...] = jnp.concatenate([i1, i2], axis=0)
    tw_ref[...] = jnp.concatenate([w1, w2], axis=0)


@jax.jit
def kernel(hidden_states, weight):
    bsz, seq_len, h = hidden_states.shape
    n = bsz * seq_len
    x = hidden_states.reshape(n, h)
    grid = (n // _BLK,)
    idx_t, tw_t = pl.pallas_call(
        _gate_body,
        grid=grid,
        in_specs=[
            pl.BlockSpec((_BLK, h), lambda i: (i, 0)),
            pl.BlockSpec((_N_EXPERTS, h), lambda i: (0, 0)),
        ],
        out_specs=[
            pl.BlockSpec((_TOP_K, _BLK), lambda i: (0, i)),
            pl.BlockSpec((_TOP_K, _BLK), lambda i: (0, i)),
        ],
        out_shape=[
            jax.ShapeDtypeStruct((_TOP_K, n), jnp.int32),
            jax.ShapeDtypeStruct((_TOP_K, n), jnp.float32),
        ],
    )(x, weight)
    return idx_t.T, tw_t.T


# speedup vs baseline: 2.5368x; 1.0041x over previous
import jax
import jax.numpy as jnp
from jax.experimental import pallas as pl

_TOP_K = 2
_N_EXPERTS = 8
_BLK = 4096


def _gate_body(x_ref, w_ref, idx_ref, tw_ref):
    x = x_ref[...]
    w = w_ref[...]
    logits = jax.lax.dot_general(
        w, x, (((1,), (1,)), ((), ())), preferred_element_type=jnp.float32
    )

    iota = jax.lax.broadcasted_iota(jnp.int32, logits.shape, 0)
    l1 = jnp.max(logits, axis=0, keepdims=True)
    i1 = jnp.min(jnp.where(logits == l1, iota, _N_EXPERTS), axis=0, keepdims=True)
    masked = jnp.where(iota == i1, -jnp.inf, logits)
    l2 = jnp.max(masked, axis=0, keepdims=True)
    i2 = jnp.min(jnp.where(masked == l2, iota, _N_EXPERTS), axis=0, keepdims=True)

    unnorm = jnp.exp(logits - l1)
    denom = jnp.sum(unnorm, axis=0, keepdims=True)
    w1 = jnp.float32(1.0) / denom
    w2 = jnp.exp(l2 - l1) / denom

    idx_ref[...] = jnp.concatenate([i1, i2], axis=0)
    tw_ref[...] = jnp.concatenate([w1, w2], axis=0)


@jax.jit
def kernel(hidden_states, weight):
    bsz, seq_len, h = hidden_states.shape
    n = bsz * seq_len
    x = hidden_states.reshape(n, h)
    grid = (n // _BLK,)
    idx_t, tw_t = pl.pallas_call(
        _gate_body,
        grid=grid,
        in_specs=[
            pl.BlockSpec((_BLK, h), lambda i: (i, 0)),
            pl.BlockSpec((_N_EXPERTS, h), lambda i: (0, 0)),
        ],
        out_specs=[
            pl.BlockSpec((_TOP_K, _BLK), lambda i: (0, i)),
            pl.BlockSpec((_TOP_K, _BLK), lambda i: (0, i)),
        ],
        out_shape=[
            jax.ShapeDtypeStruct((_TOP_K, n), jnp.int32),
            jax.ShapeDtypeStruct((_TOP_K, n), jnp.float32),
        ],
    )(x, weight)
    return idx_t.T, tw_t.T
